# baseline (speedup 1.0000x reference)
import functools

import jax
import jax.numpy as jnp
from jax import lax
from jax.experimental import pallas as pl
from jax.experimental.pallas import tpu as pltpu
from jax.experimental.pallas import tpu_sc as plsc

NC = 2
NS = 16
NW = NC * NS
L = 16

BATCH = 16384
D = 64
B_PER_W = BATCH // NW
CHUNK = 128
NCHUNK = B_PER_W // CHUNK
NG = CHUNK // L


def _fcf_body(user_hbm, item_hbm, utabT_hbm, itabT_hbm, params_hbm, out_hbm,
              uidx_v, iidx_v, ubuf_v, ibuf_v, params_v, wsplat_v, out_v,
              usem, isem):
    wid = lax.axis_index("s") * NC + lax.axis_index("c")
    base = wid * B_PER_W

    pltpu.sync_copy(user_hbm.at[pl.ds(base, B_PER_W)], uidx_v)
    pltpu.sync_copy(item_hbm.at[pl.ds(base, B_PER_W)], iidx_v)
    pltpu.sync_copy(params_hbm, params_v)

    for c in range(D // L):
        wv = params_v[pl.ds(c * L, L)]
        for l in range(L):
            wsplat_v[c * L + l, :] = jnp.full((L,), wv[l], jnp.float32)
    bias_splat = jnp.full((L,), params_v[pl.ds(D, L)][0], jnp.float32)

    def chunk_body(j, carry):
        uidx_sl = uidx_v.at[pl.ds(j * CHUNK, CHUNK)]
        iidx_sl = iidx_v.at[pl.ds(j * CHUNK, CHUNK)]
        copies = []
        for d in range(D):
            copies.append(pltpu.async_copy(
                utabT_hbm.at[d].at[uidx_sl], ubuf_v.at[d], usem))
            copies.append(pltpu.async_copy(
                itabT_hbm.at[d].at[iidx_sl], ibuf_v.at[d], isem))
        for cp in copies:
            cp.wait()

        def d_body(d, accs):
            w_d = wsplat_v[d, :]
            return tuple(
                accs[g] + ubuf_v[d, pl.ds(g * L, L)]
                * ibuf_v[d, pl.ds(g * L, L)] * w_d
                for g in range(NG)
            )

        zero = jnp.zeros((L,), jnp.float32)
        accs = lax.fori_loop(0, D, d_body, (zero,) * NG)
        for g in range(NG):
            out_v[pl.ds(j * CHUNK + g * L, L)] = accs[g] + bias_splat
        return carry

    lax.fori_loop(0, NCHUNK, chunk_body, 0)

    pltpu.sync_copy(out_v, out_hbm.at[pl.ds(base, B_PER_W)])


def kernel(user, item, users_embeddings, items_embeddings, affine_w, affine_b):
    user_i = user.astype(jnp.int32)
    item_i = item.astype(jnp.int32)
    utabT = users_embeddings.T
    itabT = items_embeddings.T
    params = jnp.concatenate(
        [affine_w.reshape(-1), affine_b.reshape(-1),
         jnp.zeros((15,), jnp.float32)])

    mesh = plsc.VectorSubcoreMesh(core_axis_name="c", subcore_axis_name="s")
    fcf = functools.partial(
        pl.kernel,
        mesh=mesh,
        compiler_params=pltpu.CompilerParams(
            needs_layout_passes=False, use_tc_tiling_on_sc=False),
        out_type=jax.ShapeDtypeStruct((BATCH,), jnp.float32),
        scratch_types=[
            pltpu.VMEM((B_PER_W,), jnp.int32),
            pltpu.VMEM((B_PER_W,), jnp.int32),
            pltpu.VMEM((D, CHUNK), jnp.float32),
            pltpu.VMEM((D, CHUNK), jnp.float32),
            pltpu.VMEM((80,), jnp.float32),
            pltpu.VMEM((D, L), jnp.float32),
            pltpu.VMEM((B_PER_W,), jnp.float32),
            pltpu.SemaphoreType.DMA,
            pltpu.SemaphoreType.DMA,
        ],
    )(_fcf_body)
    return fcf(user_i, item_i, utabT, itabT, params)


# speedup vs baseline: 9.0900x; 9.0900x over previous
import functools

import jax
import jax.numpy as jnp
from jax import lax
from jax.experimental import pallas as pl
from jax.experimental.pallas import tpu as pltpu
from jax.experimental.pallas import tpu_sc as plsc

NC = 2
NS = 16
NW = NC * NS
L = 16

NROWS = 1000000
BATCH = 16384
D = 64
W2 = 2 * D
B_PER_W = BATCH // NW
CHUNK = 128
NCHUNK = B_PER_W // CHUNK
NG = CHUNK // L


def _fcf_body(user_hbm, item_hbm, utab_hbm, itab_hbm, params_hbm, out_hbm,
              uidx_v, iidx_v, ublk_v, iblk_v, ubuf_v, ibuf_v, params_v,
              out_v, mat_v, sems):
    wid = lax.axis_index("s") * NC + lax.axis_index("c")
    base = wid * B_PER_W

    pltpu.sync_copy(user_hbm.at[pl.ds(base, B_PER_W)], uidx_v)
    pltpu.sync_copy(item_hbm.at[pl.ds(base, B_PER_W)], iidx_v)
    pltpu.sync_copy(params_hbm, params_v)

    for g in range(B_PER_W // L):
        sl = pl.ds(g * L, L)
        ublk_v[sl] = lax.shift_right_logical(uidx_v[sl], 1)
        iblk_v[sl] = lax.shift_right_logical(iidx_v[sl], 1)

    def fire(j, slot):
        pltpu.async_copy(
            utab_hbm.at[ublk_v.at[pl.ds(j * CHUNK, CHUNK)]],
            ubuf_v.at[slot], sems.at[slot, 0])
        pltpu.async_copy(
            itab_hbm.at[iblk_v.at[pl.ds(j * CHUNK, CHUNK)]],
            ibuf_v.at[slot], sems.at[slot, 1])

    def drain(slot):
        pltpu.make_async_copy(
            utab_hbm.at[ublk_v.at[pl.ds(0, CHUNK)]],
            ubuf_v.at[slot], sems.at[slot, 0]).wait()
        pltpu.make_async_copy(
            itab_hbm.at[iblk_v.at[pl.ds(0, CHUNK)]],
            ibuf_v.at[slot], sems.at[slot, 1]).wait()

    w0 = params_v[pl.ds(0, L)]
    w1 = params_v[pl.ds(L, L)]
    w2 = params_v[pl.ds(2 * L, L)]
    w3 = params_v[pl.ds(3 * L, L)]
    bias_splat = jnp.full((L,), params_v[pl.ds(D, L)][0], jnp.float32)
    iota = lax.iota(jnp.int32, L)
    one = jnp.full((L,), 1, jnp.int32)

    def compute(j, slot):
        def grp(g, carry):
            sl = pl.ds(j * CHUNK + g * L, L)
            uoffv = lax.bitwise_and(uidx_v[sl], one) * D
            ioffv = lax.bitwise_and(iidx_v[sl], one) * D
            for bb in range(L):
                b = g * L + bb
                uo = uoffv[bb]
                io = ioffv[bb]
                acc = (ubuf_v[slot, b, pl.ds(uo, L)]
                       * ibuf_v[slot, b, pl.ds(io, L)] * w0)
                acc = acc + (ubuf_v[slot, b, pl.ds(uo + L, L)]
                             * ibuf_v[slot, b, pl.ds(io + L, L)] * w1)
                acc = acc + (ubuf_v[slot, b, pl.ds(uo + 2 * L, L)]
                             * ibuf_v[slot, b, pl.ds(io + 2 * L, L)] * w2)
                acc = acc + (ubuf_v[slot, b, pl.ds(uo + 3 * L, L)]
                             * ibuf_v[slot, b, pl.ds(io + 3 * L, L)] * w3)
                mat_v[bb, :] = acc
            colsum = bias_splat
            for c in range(L):
                colsum = colsum + plsc.load_gather(
                    mat_v, [iota, jnp.full((L,), c, jnp.int32)])
            out_v[pl.ds(j * CHUNK + g * L, L)] = colsum
            return carry

        lax.fori_loop(0, NG, grp, 0)

    fire(0, 0)
    for j in range(NCHUNK):
        slot = j % 2
        if j + 1 < NCHUNK:
            fire(j + 1, 1 - slot)
        drain(slot)
        compute(j, slot)

    pltpu.sync_copy(out_v, out_hbm.at[pl.ds(base, B_PER_W)])


def kernel(user, item, users_embeddings, items_embeddings, affine_w, affine_b):
    user_i = user.astype(jnp.int32)
    item_i = item.astype(jnp.int32)
    utab2 = users_embeddings.reshape(NROWS // 2, W2)
    itab2 = items_embeddings.reshape(NROWS // 2, W2)
    params = jnp.concatenate(
        [affine_w.reshape(-1), affine_b.reshape(-1),
         jnp.zeros((15,), jnp.float32)])

    mesh = plsc.VectorSubcoreMesh(core_axis_name="c", subcore_axis_name="s")
    fcf = functools.partial(
        pl.kernel,
        mesh=mesh,
        compiler_params=pltpu.CompilerParams(
            needs_layout_passes=False, use_tc_tiling_on_sc=False),
        out_type=jax.ShapeDtypeStruct((BATCH,), jnp.float32),
        scratch_types=[
            pltpu.VMEM((B_PER_W,), jnp.int32),
            pltpu.VMEM((B_PER_W,), jnp.int32),
            pltpu.VMEM((B_PER_W,), jnp.int32),
            pltpu.VMEM((B_PER_W,), jnp.int32),
            pltpu.VMEM((2, CHUNK, W2), jnp.float32),
            pltpu.VMEM((2, CHUNK, W2), jnp.float32),
            pltpu.VMEM((80,), jnp.float32),
            pltpu.VMEM((B_PER_W,), jnp.float32),
            pltpu.VMEM((L, L), jnp.float32),
            pltpu.SemaphoreType.DMA((2, 2)),
        ],
    )(_fcf_body)
    return fcf(user_i, item_i, utab2, itab2, params)
